# Initial kernel scaffold; baseline (speedup 1.0000x reference)
import functools

import jax
import jax.numpy as jnp
from jax import lax
from jax.experimental import pallas as pl
from jax.experimental.pallas import tpu as pltpu
import jax.experimental.pallas.tpu_sc as plsc

T = 4096
N = 16
C = 256
T2 = T // 2
CHUNK = 128
NCHUNKS = T2 // CHUNK


def _body(x_hbm, lens_hbm, out_hbm, lens_out_hbm, buf, zbuf, lens_v, lo_v):
    n = lax.axis_index("s")
    j = lax.axis_index("c")

    pltpu.sync_copy(lens_hbm, lens_v)
    l = lens_v[...]
    cnt_vec = (l + (1 - j)) // 2
    lane = lax.iota(jnp.int32, 16)
    cnt = jnp.sum(jnp.where(lane == n, cnt_vec, 0))

    @pl.when(jnp.logical_and(n == 0, j == 0))
    def _():
        lo_v[...] = (l + 1) // 2
        pltpu.sync_copy(lo_v, lens_out_hbm)

    zvec = jnp.zeros((16,), jnp.float32)

    def _zrow(r, carry):
        for v in range(C // 16):
            zbuf[r, pl.ds(v * 16, 16)] = zvec
        return carry

    lax.fori_loop(0, CHUNK, _zrow, 0)

    qi = j * 16 + n
    qo = n * 2 + j

    def _chunk(i, carry):
        t2s = i * CHUNK

        @pl.when(cnt <= t2s)
        def _():
            pltpu.sync_copy(zbuf, out_hbm.at[pl.ds(t2s, CHUNK), qo, :])

        @pl.when(cnt > t2s)
        def _():
            pltpu.sync_copy(x_hbm.at[pl.ds(t2s, CHUNK), qi, :], buf)

            @pl.when(cnt < t2s + CHUNK)
            def _():
                m = cnt - t2s

                def _mrow(r, c2):
                    @pl.when(r >= m)
                    def _():
                        for v in range(C // 16):
                            buf[r, pl.ds(v * 16, 16)] = zvec

                    return c2

                lax.fori_loop(0, CHUNK, _mrow, 0)

            pltpu.sync_copy(buf, out_hbm.at[pl.ds(t2s, CHUNK), qo, :])

        return carry

    lax.fori_loop(0, NCHUNKS, _chunk, 0)


@jax.jit
def kernel(x, x_lens):
    x3 = x.reshape(T2, 2 * N, C)
    mesh = plsc.VectorSubcoreMesh(core_axis_name="c", subcore_axis_name="s")
    out3, lens_out = pl.kernel(
        _body,
        out_type=[
            jax.ShapeDtypeStruct((T2, 2 * N, C), jnp.float32),
            jax.ShapeDtypeStruct((N,), jnp.int32),
        ],
        mesh=mesh,
        scratch_types=[
            pltpu.VMEM((CHUNK, C), jnp.float32),
            pltpu.VMEM((CHUNK, C), jnp.float32),
            pltpu.VMEM((N,), jnp.int32),
            pltpu.VMEM((N,), jnp.int32),
        ],
    )(x3, x_lens.astype(jnp.int32))
    return out3.reshape(T2, N, 2 * C), lens_out


# speedup vs baseline: 1.5052x; 1.5052x over previous
import jax
import jax.numpy as jnp
from jax import lax
from jax.experimental import pallas as pl
from jax.experimental.pallas import tpu as pltpu
import jax.experimental.pallas.tpu_sc as plsc

T = 4096
N = 16
C = 256
T2 = T // 2
CHUNK = 128
NCHUNKS = T2 // CHUNK


def _body(x_hbm, lens_hbm, out_hbm, lens_out_hbm, buf, zbuf, lens_v, lo_v):
    n = lax.axis_index("s")
    j = lax.axis_index("c")

    pltpu.sync_copy(lens_hbm, lens_v.at[pl.ds(0, N)])
    pltpu.sync_copy(lens_hbm, lens_v.at[pl.ds(N, N)])
    myl = lens_v[pl.ds(n, 16)][0]
    cnt = lax.shift_right_logical(myl + (1 - j), 1)

    @pl.when(jnp.logical_and(n == 0, j == 0))
    def _():
        l = lens_v[pl.ds(0, 16)]
        lo_v[...] = lax.shift_right_logical(l + 1, 1)
        pltpu.sync_copy(lo_v, lens_out_hbm)

    zvec = jnp.zeros((16,), jnp.float32)

    def _zrow(r, carry):
        for v in range(C // 16):
            zbuf[r, pl.ds(v * 16, 16)] = zvec
        return carry

    lax.fori_loop(0, CHUNK, _zrow, 0)

    qi = j * 16 + n
    qo = n * 2 + j

    def _chunk(i, carry):
        t2s = i * CHUNK

        @pl.when(cnt <= t2s)
        def _():
            pltpu.sync_copy(zbuf, out_hbm.at[pl.ds(t2s, CHUNK), qo, :])

        @pl.when(cnt > t2s)
        def _():
            pltpu.sync_copy(x_hbm.at[pl.ds(t2s, CHUNK), qi, :], buf)

            @pl.when(cnt < t2s + CHUNK)
            def _():
                m = cnt - t2s

                def _mrow(r, c2):
                    @pl.when(r >= m)
                    def _():
                        for v in range(C // 16):
                            buf[r, pl.ds(v * 16, 16)] = zvec

                    return c2

                lax.fori_loop(0, CHUNK, _mrow, 0)

            pltpu.sync_copy(buf, out_hbm.at[pl.ds(t2s, CHUNK), qo, :])

        return carry

    lax.fori_loop(0, NCHUNKS, _chunk, 0)


@jax.jit
def kernel(x, x_lens):
    x3 = x.reshape(T2, 2 * N, C)
    mesh = plsc.VectorSubcoreMesh(core_axis_name="c", subcore_axis_name="s")
    out3, lens_out = pl.kernel(
        _body,
        out_type=[
            jax.ShapeDtypeStruct((T2, 2 * N, C), jnp.float32),
            jax.ShapeDtypeStruct((N,), jnp.int32),
        ],
        mesh=mesh,
        scratch_types=[
            pltpu.VMEM((CHUNK, C), jnp.float32),
            pltpu.VMEM((CHUNK, C), jnp.float32),
            pltpu.VMEM((2 * N,), jnp.int32),
            pltpu.VMEM((N,), jnp.int32),
        ],
    )(x3, x_lens.astype(jnp.int32))
    return out3.reshape(T2, N, 2 * C), lens_out
